# Initial kernel scaffold; baseline (speedup 1.0000x reference)
import jax
import jax.numpy as jnp
from jax.experimental import pallas as pl


def kernel(features, adj, W_e1, W_mean, W_std, Wl0_a, Wl1_a, Wl2_a, Wl0_b, Wl1_b, Wl2_b, W_h1, W_h2, W_out):
    raise NotImplementedError("write your pallas kernel here")


# speedup vs baseline: 1.1319x; 1.1319x over previous
import functools

import jax
import jax.numpy as jnp
from jax.experimental import pallas as pl

N = 4096
BM = 256
_AR = 0.5
f32 = jnp.float32


def _dot_t(a, b):
    return jax.lax.dot_general(a, b, (((1,), (1,)), ((), ())),
                               preferred_element_type=f32)


def _proj_body(x_ref, w_ref, o_ref):
    o_ref[...] = jnp.dot(x_ref[...], w_ref[...], preferred_element_type=f32)


def _proj(x, w):
    n, k = x.shape
    c = w.shape[1]
    return pl.pallas_call(
        _proj_body,
        grid=(n // BM,),
        in_specs=[pl.BlockSpec((BM, k), lambda i: (i, 0)),
                  pl.BlockSpec((k, c), lambda i: (0, 0))],
        out_specs=pl.BlockSpec((BM, c), lambda i: (i, 0)),
        out_shape=jax.ShapeDtypeStruct((n, c), f32),
    )(x, w)


def _adj_body(do_relu, has_w, has_add, *refs):
    adj_ref, m_ref = refs[0], refs[1]
    idx = 2
    w_ref = add_ref = None
    if has_w:
        w_ref = refs[idx]
        idx += 1
    if has_add:
        add_ref = refs[idx]
        idx += 1
    o_ref = refs[idx]
    acc = jnp.dot(adj_ref[...], m_ref[...], preferred_element_type=f32)
    if has_w:
        acc = jnp.dot(acc, w_ref[...], preferred_element_type=f32)
    if do_relu:
        acc = jnp.maximum(acc, 0.0)
    if has_add:
        acc = acc + add_ref[...]
    o_ref[...] = acc


def _adj_pass(adj, m, post_w=None, relu=False, add=None):
    n = adj.shape[0]
    c = m.shape[1]
    cout = post_w.shape[1] if post_w is not None else c
    ins = [adj, m]
    specs = [pl.BlockSpec((BM, n), lambda i: (i, 0)),
             pl.BlockSpec((n, c), lambda i: (0, 0))]
    if post_w is not None:
        ins.append(post_w)
        specs.append(pl.BlockSpec(post_w.shape, lambda i: (0, 0)))
    if add is not None:
        ins.append(add)
        specs.append(pl.BlockSpec((BM, cout), lambda i: (i, 0)))
    return pl.pallas_call(
        functools.partial(_adj_body, relu, post_w is not None, add is not None),
        grid=(n // BM,),
        in_specs=specs,
        out_specs=pl.BlockSpec((BM, cout), lambda i: (i, 0)),
        out_shape=jax.ShapeDtypeStruct((n, cout), f32),
    )(*ins)


def _phase_a_body(z_blk, z_all, xw0, w1a, w1b, rs_ref, m_ref):
    zi = z_blk[...]
    sg = jax.nn.sigmoid(_dot_t(zi, z_all[...]))
    rs_ref[...] = jnp.sum(sg, axis=1, keepdims=True)
    m_ref[...] = jnp.concatenate(
        [jnp.dot(zi, w1a[...], preferred_element_type=f32),
         xw0[..., :32],
         jnp.dot(zi, w1b[...], preferred_element_type=f32),
         xw0[..., 32:]], axis=1)


def _phase_a(z, xw0, w1a, w1b):
    return pl.pallas_call(
        _phase_a_body,
        grid=(N // BM,),
        in_specs=[pl.BlockSpec((BM, 16), lambda i: (i, 0)),
                  pl.BlockSpec((N, 16), lambda i: (0, 0)),
                  pl.BlockSpec((BM, 64), lambda i: (i, 0)),
                  pl.BlockSpec((16, 32), lambda i: (0, 0)),
                  pl.BlockSpec((16, 32), lambda i: (0, 0))],
        out_specs=[pl.BlockSpec((BM, 1), lambda i: (i, 0)),
                   pl.BlockSpec((BM, 128), lambda i: (i, 0))],
        out_shape=[jax.ShapeDtypeStruct((N, 1), f32),
                   jax.ShapeDtypeStruct((N, 128), f32)],
    )(z, z, xw0, w1a, w1b)


def _phase_b_body(z_blk, z_all, rs_all, rs_blk, m_all, w2a, w2b, o_ref):
    d_all = jax.lax.rsqrt(rs_all[...])
    md = m_all[...] * d_all
    sg = jax.nn.sigmoid(_dot_t(z_blk[...], z_all[...]))
    acc = jnp.dot(sg, md, preferred_element_type=f32)
    sc = acc * jax.lax.rsqrt(rs_blk[...])
    ua = jnp.maximum(sc[:, 0:32], 0.0) + jnp.maximum(sc[:, 32:64], 0.0)
    ub = jnp.maximum(sc[:, 64:96], 0.0) + jnp.maximum(sc[:, 96:128], 0.0)
    o_ref[...] = jnp.concatenate(
        [jnp.dot(ua, w2a[...], preferred_element_type=f32),
         jnp.dot(ub, w2b[...], preferred_element_type=f32)], axis=1)


def _phase_b(z, rs, m, w2a, w2b):
    return pl.pallas_call(
        _phase_b_body,
        grid=(N // BM,),
        in_specs=[pl.BlockSpec((BM, 16), lambda i: (i, 0)),
                  pl.BlockSpec((N, 16), lambda i: (0, 0)),
                  pl.BlockSpec((N, 1), lambda i: (0, 0)),
                  pl.BlockSpec((BM, 1), lambda i: (i, 0)),
                  pl.BlockSpec((N, 128), lambda i: (0, 0)),
                  pl.BlockSpec((32, 16), lambda i: (0, 0)),
                  pl.BlockSpec((32, 16), lambda i: (0, 0))],
        out_specs=pl.BlockSpec((BM, 32), lambda i: (i, 0)),
        out_shape=jax.ShapeDtypeStruct((N, 32), f32),
    )(z, z, rs, rs, m, w2a, w2b)


def _phase_c_body(z_blk, z_all, rs_all, rs_blk, v_all, o_ref):
    d_all = jax.lax.rsqrt(rs_all[...])
    vd = v_all[...] * d_all
    sg = jax.nn.sigmoid(_dot_t(z_blk[...], z_all[...]))
    acc = jnp.dot(sg, vd, preferred_element_type=f32)
    w = acc * jax.lax.rsqrt(rs_blk[...])
    zz = jnp.concatenate([z_blk[...], z_blk[...]], axis=1)
    o_ref[...] = (1.0 - _AR) * zz + _AR * w


def _phase_c(z, rs, v):
    return pl.pallas_call(
        _phase_c_body,
        grid=(N // BM,),
        in_specs=[pl.BlockSpec((BM, 16), lambda i: (i, 0)),
                  pl.BlockSpec((N, 16), lambda i: (0, 0)),
                  pl.BlockSpec((N, 1), lambda i: (0, 0)),
                  pl.BlockSpec((BM, 1), lambda i: (i, 0)),
                  pl.BlockSpec((N, 32), lambda i: (0, 0))],
        out_specs=pl.BlockSpec((BM, 32), lambda i: (i, 0)),
        out_shape=jax.ShapeDtypeStruct((N, 32), f32),
    )(z, z, rs, rs, v)


def _recon_body(u_blk, u_all, o_ref):
    o_ref[...] = _dot_t(u_blk[...], u_all[...])


def _recon(u):
    return pl.pallas_call(
        _recon_body,
        grid=(N // BM,),
        in_specs=[pl.BlockSpec((BM, 16), lambda i: (i, 0)),
                  pl.BlockSpec((N, 16), lambda i: (0, 0))],
        out_specs=pl.BlockSpec((BM, N), lambda i: (i, 0)),
        out_shape=jax.ShapeDtypeStruct((N, N), f32),
    )(u, u)


def kernel(features, adj, W_e1, W_mean, W_std, Wl0_a, Wl1_a, Wl2_a,
           Wl0_b, Wl1_b, Wl2_b, W_h1, W_h2, W_out):
    wcat = jnp.concatenate([W_e1, W_h1, Wl0_a, Wl0_b], axis=1)
    p = _proj(features, wcat)

    t1 = _adj_pass(adj, p[:, :64], relu=True)
    hidden1, h1r = t1[:, :32], t1[:, 32:]
    z = _adj_pass(adj, hidden1, post_w=W_mean)

    rs, m = _phase_a(z, p[:, 64:], Wl1_a, Wl1_b)
    v = _phase_b(z, rs, m, Wl2_a, Wl2_b)
    upd = _phase_c(z, rs, v)
    u_a, z_f = upd[:, :16], upd[:, 16:]

    t3 = _adj_pass(adj, z_f, post_w=W_h2, relu=True, add=h1r)
    outputs = _adj_pass(adj, t3, post_w=W_out)

    reconstructions = _recon(u_a).reshape(-1)
    return outputs, reconstructions


# speedup vs baseline: 1.2313x; 1.0878x over previous
import functools

import jax
import jax.numpy as jnp
from jax.experimental import pallas as pl

N = 4096
BM = 256
_AR = 0.5
f32 = jnp.float32
bf16 = jnp.bfloat16


def _dot_t(a, b):
    return jax.lax.dot_general(a, b, (((1,), (1,)), ((), ())),
                               preferred_element_type=f32)


def _proj_body(x_ref, w_ref, o_ref):
    o_ref[...] = jnp.dot(x_ref[...], w_ref[...], preferred_element_type=f32)


def _proj(x, w):
    n, k = x.shape
    c = w.shape[1]
    return pl.pallas_call(
        _proj_body,
        grid=(n // BM,),
        in_specs=[pl.BlockSpec((BM, k), lambda i: (i, 0)),
                  pl.BlockSpec((k, c), lambda i: (0, 0))],
        out_specs=pl.BlockSpec((BM, c), lambda i: (i, 0)),
        out_shape=jax.ShapeDtypeStruct((n, c), f32),
    )(x, w)


def _adj_first_body(adj_ref, m_ref, o_ref, ab_ref):
    ab = adj_ref[...].astype(bf16)
    ab_ref[...] = ab
    acc = jnp.dot(ab, m_ref[...].astype(bf16), preferred_element_type=f32)
    o_ref[...] = jnp.maximum(acc, 0.0)


def _adj_first(adj, m):
    n = adj.shape[0]
    c = m.shape[1]
    return pl.pallas_call(
        _adj_first_body,
        grid=(n // BM,),
        in_specs=[pl.BlockSpec((BM, n), lambda i: (i, 0)),
                  pl.BlockSpec((n, c), lambda i: (0, 0))],
        out_specs=[pl.BlockSpec((BM, c), lambda i: (i, 0)),
                   pl.BlockSpec((BM, n), lambda i: (i, 0))],
        out_shape=[jax.ShapeDtypeStruct((n, c), f32),
                   jax.ShapeDtypeStruct((n, n), bf16)],
    )(adj, m)


def _adj_body(do_relu, has_w, has_add, *refs):
    adj_ref, m_ref = refs[0], refs[1]
    idx = 2
    w_ref = add_ref = None
    if has_w:
        w_ref = refs[idx]
        idx += 1
    if has_add:
        add_ref = refs[idx]
        idx += 1
    o_ref = refs[idx]
    acc = jnp.dot(adj_ref[...], m_ref[...].astype(bf16),
                  preferred_element_type=f32)
    if has_w:
        acc = jnp.dot(acc, w_ref[...], preferred_element_type=f32)
    if do_relu:
        acc = jnp.maximum(acc, 0.0)
    if has_add:
        acc = acc + add_ref[...]
    o_ref[...] = acc


def _adj_pass(adj_b, m, post_w=None, relu=False, add=None):
    n = adj_b.shape[0]
    c = m.shape[1]
    cout = post_w.shape[1] if post_w is not None else c
    ins = [adj_b, m]
    specs = [pl.BlockSpec((BM, n), lambda i: (i, 0)),
             pl.BlockSpec((n, c), lambda i: (0, 0))]
    if post_w is not None:
        ins.append(post_w)
        specs.append(pl.BlockSpec(post_w.shape, lambda i: (0, 0)))
    if add is not None:
        ins.append(add)
        specs.append(pl.BlockSpec((BM, cout), lambda i: (i, 0)))
    return pl.pallas_call(
        functools.partial(_adj_body, relu, post_w is not None, add is not None),
        grid=(n // BM,),
        in_specs=specs,
        out_specs=pl.BlockSpec((BM, cout), lambda i: (i, 0)),
        out_shape=jax.ShapeDtypeStruct((n, cout), f32),
    )(*ins)


def _phase_a_body(z_blk, z_all, xw0, w1a, w1b, rs_ref, m_ref):
    zi = z_blk[...]
    sg = jax.nn.sigmoid(_dot_t(zi, z_all[...]))
    rs_ref[...] = jnp.sum(sg, axis=1, keepdims=True)
    m_ref[...] = jnp.concatenate(
        [jnp.dot(zi, w1a[...], preferred_element_type=f32),
         xw0[..., :32],
         jnp.dot(zi, w1b[...], preferred_element_type=f32),
         xw0[..., 32:]], axis=1)


def _phase_a(z, xw0, w1a, w1b):
    return pl.pallas_call(
        _phase_a_body,
        grid=(N // BM,),
        in_specs=[pl.BlockSpec((BM, 16), lambda i: (i, 0)),
                  pl.BlockSpec((N, 16), lambda i: (0, 0)),
                  pl.BlockSpec((BM, 64), lambda i: (i, 0)),
                  pl.BlockSpec((16, 32), lambda i: (0, 0)),
                  pl.BlockSpec((16, 32), lambda i: (0, 0))],
        out_specs=[pl.BlockSpec((BM, 1), lambda i: (i, 0)),
                   pl.BlockSpec((BM, 128), lambda i: (i, 0))],
        out_shape=[jax.ShapeDtypeStruct((N, 1), f32),
                   jax.ShapeDtypeStruct((N, 128), f32)],
    )(z, z, xw0, w1a, w1b)


def _phase_b_body(z_blk, z_all, rs_all, rs_blk, m_all, w2a, w2b, o_ref):
    d_all = jax.lax.rsqrt(rs_all[...])
    md = (m_all[...] * d_all).astype(bf16)
    sg = jax.nn.sigmoid(_dot_t(z_blk[...], z_all[...]))
    acc = jnp.dot(sg.astype(bf16), md, preferred_element_type=f32)
    sc = acc * jax.lax.rsqrt(rs_blk[...])
    ua = jnp.maximum(sc[:, 0:32], 0.0) + jnp.maximum(sc[:, 32:64], 0.0)
    ub = jnp.maximum(sc[:, 64:96], 0.0) + jnp.maximum(sc[:, 96:128], 0.0)
    o_ref[...] = jnp.concatenate(
        [jnp.dot(ua, w2a[...], preferred_element_type=f32),
         jnp.dot(ub, w2b[...], preferred_element_type=f32)], axis=1)


def _phase_b(z, rs, m, w2a, w2b):
    return pl.pallas_call(
        _phase_b_body,
        grid=(N // BM,),
        in_specs=[pl.BlockSpec((BM, 16), lambda i: (i, 0)),
                  pl.BlockSpec((N, 16), lambda i: (0, 0)),
                  pl.BlockSpec((N, 1), lambda i: (0, 0)),
                  pl.BlockSpec((BM, 1), lambda i: (i, 0)),
                  pl.BlockSpec((N, 128), lambda i: (0, 0)),
                  pl.BlockSpec((32, 16), lambda i: (0, 0)),
                  pl.BlockSpec((32, 16), lambda i: (0, 0))],
        out_specs=pl.BlockSpec((BM, 32), lambda i: (i, 0)),
        out_shape=jax.ShapeDtypeStruct((N, 32), f32),
    )(z, z, rs, rs, m, w2a, w2b)


def _phase_c_body(z_blk, z_all, rs_all, rs_blk, v_all, o_ref):
    d_all = jax.lax.rsqrt(rs_all[...])
    vd = (v_all[...] * d_all).astype(bf16)
    sg = jax.nn.sigmoid(_dot_t(z_blk[...], z_all[...]))
    acc = jnp.dot(sg.astype(bf16), vd, preferred_element_type=f32)
    w = acc * jax.lax.rsqrt(rs_blk[...])
    zz = jnp.concatenate([z_blk[...], z_blk[...]], axis=1)
    o_ref[...] = (1.0 - _AR) * zz + _AR * w


def _phase_c(z, rs, v):
    return pl.pallas_call(
        _phase_c_body,
        grid=(N // BM,),
        in_specs=[pl.BlockSpec((BM, 16), lambda i: (i, 0)),
                  pl.BlockSpec((N, 16), lambda i: (0, 0)),
                  pl.BlockSpec((N, 1), lambda i: (0, 0)),
                  pl.BlockSpec((BM, 1), lambda i: (i, 0)),
                  pl.BlockSpec((N, 32), lambda i: (0, 0))],
        out_specs=pl.BlockSpec((BM, 32), lambda i: (i, 0)),
        out_shape=jax.ShapeDtypeStruct((N, 32), f32),
    )(z, z, rs, rs, v)


def _recon_body(u_blk, u_all, o_ref):
    o_ref[...] = _dot_t(u_blk[...], u_all[...])


def _recon(u):
    return pl.pallas_call(
        _recon_body,
        grid=(N // BM,),
        in_specs=[pl.BlockSpec((BM, 16), lambda i: (i, 0)),
                  pl.BlockSpec((N, 16), lambda i: (0, 0))],
        out_specs=pl.BlockSpec((BM, N), lambda i: (i, 0)),
        out_shape=jax.ShapeDtypeStruct((N, N), f32),
    )(u, u)


def kernel(features, adj, W_e1, W_mean, W_std, Wl0_a, Wl1_a, Wl2_a,
           Wl0_b, Wl1_b, Wl2_b, W_h1, W_h2, W_out):
    wcat = jnp.concatenate([W_e1, W_h1, Wl0_a, Wl0_b], axis=1)
    p = _proj(features, wcat)

    t1, adj_b = _adj_first(adj, p[:, :64])
    hidden1, h1r = t1[:, :32], t1[:, 32:]
    z = _adj_pass(adj_b, hidden1, post_w=W_mean)

    rs, m = _phase_a(z, p[:, 64:], Wl1_a, Wl1_b)
    v = _phase_b(z, rs, m, Wl2_a, Wl2_b)
    upd = _phase_c(z, rs, v)
    u_a, z_f = upd[:, :16], upd[:, 16:]

    reconstructions = _recon(u_a).reshape(-1)

    t3 = _adj_pass(adj_b, z_f, post_w=W_h2, relu=True, add=h1r)
    outputs = _adj_pass(adj_b, t3, post_w=W_out)
    return outputs, reconstructions


# speedup vs baseline: 1.5249x; 1.2385x over previous
import functools

import jax
import jax.numpy as jnp
from jax.experimental import pallas as pl
from jax.experimental.pallas import tpu as pltpu

N = 4096
BM = 256
_AR = 0.5
f32 = jnp.float32
bf16 = jnp.bfloat16


def _dot_t(a, b):
    return jax.lax.dot_general(a, b, (((1,), (1,)), ((), ())),
                               preferred_element_type=f32)


def _sig_t(z_blk, z_all):
    s = _dot_t(z_blk * 0.5, z_all)
    return 0.5 * jnp.tanh(s) + 0.5


def _proj_body(x_ref, w_ref, o_ref):
    o_ref[...] = jnp.dot(x_ref[...], w_ref[...], preferred_element_type=f32)


def _proj(x, w):
    n, k = x.shape
    c = w.shape[1]
    return pl.pallas_call(
        _proj_body,
        grid=(n // BM,),
        in_specs=[pl.BlockSpec((BM, k), lambda i: (i, 0)),
                  pl.BlockSpec((k, c), lambda i: (0, 0))],
        out_specs=pl.BlockSpec((BM, c), lambda i: (i, 0)),
        out_shape=jax.ShapeDtypeStruct((n, c), f32),
    )(x, w)


def _adj_first_body(adj_ref, m_ref, o_ref, ab_ref):
    ab = adj_ref[...].astype(bf16)
    ab_ref[...] = ab
    acc = jnp.dot(ab, m_ref[...].astype(bf16), preferred_element_type=f32)
    o_ref[...] = jnp.maximum(acc, 0.0)


def _adj_first(adj, m):
    n = adj.shape[0]
    c = m.shape[1]
    return pl.pallas_call(
        _adj_first_body,
        grid=(n // BM,),
        in_specs=[pl.BlockSpec((BM, n), lambda i: (i, 0)),
                  pl.BlockSpec((n, c), lambda i: (0, 0))],
        out_specs=[pl.BlockSpec((BM, c), lambda i: (i, 0)),
                   pl.BlockSpec((BM, n), lambda i: (i, 0))],
        out_shape=[jax.ShapeDtypeStruct((n, c), f32),
                   jax.ShapeDtypeStruct((n, n), bf16)],
    )(adj, m)


def _adj_body(do_relu, has_w, has_add, *refs):
    adj_ref, m_ref = refs[0], refs[1]
    idx = 2
    w_ref = add_ref = None
    if has_w:
        w_ref = refs[idx]
        idx += 1
    if has_add:
        add_ref = refs[idx]
        idx += 1
    o_ref = refs[idx]
    acc = jnp.dot(adj_ref[...], m_ref[...].astype(bf16),
                  preferred_element_type=f32)
    if has_w:
        acc = jnp.dot(acc, w_ref[...], preferred_element_type=f32)
    if do_relu:
        acc = jnp.maximum(acc, 0.0)
    if has_add:
        acc = acc + add_ref[...]
    o_ref[...] = acc


def _adj_pass(adj_b, m, post_w=None, relu=False, add=None):
    n = adj_b.shape[0]
    c = m.shape[1]
    cout = post_w.shape[1] if post_w is not None else c
    ins = [adj_b, m]
    specs = [pl.BlockSpec((BM, n), lambda i: (i, 0)),
             pl.BlockSpec((n, c), lambda i: (0, 0))]
    if post_w is not None:
        ins.append(post_w)
        specs.append(pl.BlockSpec(post_w.shape, lambda i: (0, 0)))
    if add is not None:
        ins.append(add)
        specs.append(pl.BlockSpec((BM, cout), lambda i: (i, 0)))
    return pl.pallas_call(
        functools.partial(_adj_body, relu, post_w is not None, add is not None),
        grid=(n // BM,),
        in_specs=specs,
        out_specs=pl.BlockSpec((BM, cout), lambda i: (i, 0)),
        out_shape=jax.ShapeDtypeStruct((n, cout), f32),
    )(*ins)


def _phase_a_body(z_blk, z_all, xw0, w1a, w1b, rs_ref, m_ref):
    zi = z_blk[...]
    sg = _sig_t(zi, z_all[...])
    rs_ref[...] = jnp.sum(sg, axis=1, keepdims=True)
    m_ref[...] = jnp.concatenate(
        [jnp.dot(zi, w1a[...], preferred_element_type=f32),
         xw0[..., :32],
         jnp.dot(zi, w1b[...], preferred_element_type=f32),
         xw0[..., 32:]], axis=1)


def _phase_a(z, xw0, w1a, w1b):
    return pl.pallas_call(
        _phase_a_body,
        grid=(N // BM,),
        in_specs=[pl.BlockSpec((BM, 16), lambda i: (i, 0)),
                  pl.BlockSpec((N, 16), lambda i: (0, 0)),
                  pl.BlockSpec((BM, 64), lambda i: (i, 0)),
                  pl.BlockSpec((16, 32), lambda i: (0, 0)),
                  pl.BlockSpec((16, 32), lambda i: (0, 0))],
        out_specs=[pl.BlockSpec((BM, 1), lambda i: (i, 0)),
                   pl.BlockSpec((BM, 128), lambda i: (i, 0))],
        out_shape=[jax.ShapeDtypeStruct((N, 1), f32),
                   jax.ShapeDtypeStruct((N, 128), f32)],
    )(z, z, xw0, w1a, w1b)


def _phase_b_body(z_blk, z_all, rs_all, rs_blk, m_all, w2a, w2b, o_ref,
                  md_ref):
    @pl.when(pl.program_id(0) == 0)
    def _():
        md_ref[...] = (m_all[...] * jax.lax.rsqrt(rs_all[...])).astype(bf16)

    sg = _sig_t(z_blk[...], z_all[...])
    acc = jnp.dot(sg.astype(bf16), md_ref[...], preferred_element_type=f32)
    di = jax.lax.rsqrt(rs_blk[...])
    sc = acc * di
    ua = jnp.maximum(sc[:, 0:32], 0.0) + jnp.maximum(sc[:, 32:64], 0.0)
    ub = jnp.maximum(sc[:, 64:96], 0.0) + jnp.maximum(sc[:, 96:128], 0.0)
    v = jnp.concatenate(
        [jnp.dot(ua, w2a[...], preferred_element_type=f32),
         jnp.dot(ub, w2b[...], preferred_element_type=f32)], axis=1)
    o_ref[...] = (v * di).astype(bf16)


def _phase_b(z, rs, m, w2a, w2b):
    return pl.pallas_call(
        _phase_b_body,
        grid=(N // BM,),
        in_specs=[pl.BlockSpec((BM, 16), lambda i: (i, 0)),
                  pl.BlockSpec((N, 16), lambda i: (0, 0)),
                  pl.BlockSpec((N, 1), lambda i: (0, 0)),
                  pl.BlockSpec((BM, 1), lambda i: (i, 0)),
                  pl.BlockSpec((N, 128), lambda i: (0, 0)),
                  pl.BlockSpec((32, 16), lambda i: (0, 0)),
                  pl.BlockSpec((32, 16), lambda i: (0, 0))],
        out_specs=pl.BlockSpec((BM, 32), lambda i: (i, 0)),
        out_shape=jax.ShapeDtypeStruct((N, 32), bf16),
        scratch_shapes=[pltpu.VMEM((N, 128), bf16)],
    )(z, z, rs, rs, m, w2a, w2b)


def _phase_c_body(z_blk, z_all, rs_blk, vd_all, o_ref):
    sg = _sig_t(z_blk[...], z_all[...])
    acc = jnp.dot(sg.astype(bf16), vd_all[...], preferred_element_type=f32)
    w = acc * jax.lax.rsqrt(rs_blk[...])
    zz = jnp.concatenate([z_blk[...], z_blk[...]], axis=1)
    o_ref[...] = (1.0 - _AR) * zz + _AR * w


def _phase_c(z, rs, vd):
    return pl.pallas_call(
        _phase_c_body,
        grid=(N // BM,),
        in_specs=[pl.BlockSpec((BM, 16), lambda i: (i, 0)),
                  pl.BlockSpec((N, 16), lambda i: (0, 0)),
                  pl.BlockSpec((BM, 1), lambda i: (i, 0)),
                  pl.BlockSpec((N, 32), lambda i: (0, 0))],
        out_specs=pl.BlockSpec((BM, 32), lambda i: (i, 0)),
        out_shape=jax.ShapeDtypeStruct((N, 32), f32),
    )(z, z, rs, vd)


def _recon_body(u_blk, u_all, o_ref):
    o_ref[...] = _dot_t(u_blk[...], u_all[...])


def _recon(u):
    return pl.pallas_call(
        _recon_body,
        grid=(N // BM,),
        in_specs=[pl.BlockSpec((BM, 16), lambda i: (i, 0)),
                  pl.BlockSpec((N, 16), lambda i: (0, 0))],
        out_specs=pl.BlockSpec((BM, N), lambda i: (i, 0)),
        out_shape=jax.ShapeDtypeStruct((N, N), f32),
    )(u, u)


def kernel(features, adj, W_e1, W_mean, W_std, Wl0_a, Wl1_a, Wl2_a,
           Wl0_b, Wl1_b, Wl2_b, W_h1, W_h2, W_out):
    wcat = jnp.concatenate([W_e1, W_h1, Wl0_a, Wl0_b], axis=1)
    p = _proj(features, wcat)

    t1, adj_b = _adj_first(adj, p[:, :64])
    hidden1, h1r = t1[:, :32], t1[:, 32:]
    z = _adj_pass(adj_b, hidden1, post_w=W_mean)

    rs, m = _phase_a(z, p[:, 64:], Wl1_a, Wl1_b)
    v = _phase_b(z, rs, m, Wl2_a, Wl2_b)
    upd = _phase_c(z, rs, v)
    u_a, z_f = upd[:, :16], upd[:, 16:]

    reconstructions = _recon(u_a).reshape(-1)

    t3 = _adj_pass(adj_b, z_f, post_w=W_h2, relu=True, add=h1r)
    outputs = _adj_pass(adj_b, t3, post_w=W_out)
    return outputs, reconstructions


# speedup vs baseline: 1.5544x; 1.0194x over previous
import functools

import jax
import jax.numpy as jnp
from jax.experimental import pallas as pl
from jax.experimental.pallas import tpu as pltpu

N = 4096
BM = 256
_AR = 0.5
f32 = jnp.float32
bf16 = jnp.bfloat16


def _dot_t(a, b):
    return jax.lax.dot_general(a, b, (((1,), (1,)), ((), ())),
                               preferred_element_type=f32)


def _sig_t(z_blk, z_all):
    s = _dot_t((z_blk * 0.5).astype(bf16), z_all.astype(bf16))
    return 0.5 * jnp.tanh(s) + 0.5


def _proj_body(x_ref, w_ref, o_ref):
    o_ref[...] = jnp.dot(x_ref[...], w_ref[...], preferred_element_type=f32)


def _proj(x, w):
    n, k = x.shape
    c = w.shape[1]
    return pl.pallas_call(
        _proj_body,
        grid=(n // BM,),
        in_specs=[pl.BlockSpec((BM, k), lambda i: (i, 0)),
                  pl.BlockSpec((k, c), lambda i: (0, 0))],
        out_specs=pl.BlockSpec((BM, c), lambda i: (i, 0)),
        out_shape=jax.ShapeDtypeStruct((n, c), f32),
    )(x, w)


def _adj_first_body(adj_ref, m_ref, o_ref, ab_ref):
    ab = adj_ref[...].astype(bf16)
    ab_ref[...] = ab
    acc = jnp.dot(ab, m_ref[...].astype(bf16), preferred_element_type=f32)
    o_ref[...] = jnp.maximum(acc, 0.0)


def _adj_first(adj, m):
    n = adj.shape[0]
    c = m.shape[1]
    return pl.pallas_call(
        _adj_first_body,
        grid=(n // BM,),
        in_specs=[pl.BlockSpec((BM, n), lambda i: (i, 0)),
                  pl.BlockSpec((n, c), lambda i: (0, 0))],
        out_specs=[pl.BlockSpec((BM, c), lambda i: (i, 0)),
                   pl.BlockSpec((BM, n), lambda i: (i, 0))],
        out_shape=[jax.ShapeDtypeStruct((n, c), f32),
                   jax.ShapeDtypeStruct((n, n), bf16)],
    )(adj, m)


def _adj_body(do_relu, has_w, *refs):
    adj_ref, m_ref = refs[0], refs[1]
    idx = 2
    w_ref = None
    if has_w:
        w_ref = refs[idx]
        idx += 1
    o_ref = refs[idx]
    acc = jnp.dot(adj_ref[...], m_ref[...].astype(bf16),
                  preferred_element_type=f32)
    if has_w:
        acc = jnp.dot(acc, w_ref[...], preferred_element_type=f32)
    if do_relu:
        acc = jnp.maximum(acc, 0.0)
    o_ref[...] = acc


def _adj_pass(adj_b, m, post_w=None, relu=False):
    n = adj_b.shape[0]
    c = m.shape[1]
    cout = post_w.shape[1] if post_w is not None else c
    ins = [adj_b, m]
    specs = [pl.BlockSpec((BM, n), lambda i: (i, 0)),
             pl.BlockSpec((n, c), lambda i: (0, 0))]
    if post_w is not None:
        ins.append(post_w)
        specs.append(pl.BlockSpec(post_w.shape, lambda i: (0, 0)))
    return pl.pallas_call(
        functools.partial(_adj_body, relu, post_w is not None),
        grid=(n // BM,),
        in_specs=specs,
        out_specs=pl.BlockSpec((BM, cout), lambda i: (i, 0)),
        out_shape=jax.ShapeDtypeStruct((n, cout), f32),
    )(*ins)


def _head_body(adj_ref, zf_ref, h1r_ref, wh2_ref, wout_ref, o_ref, t3_ref):
    p = pl.program_id(0)
    i = pl.program_id(1)

    @pl.when(p == 0)
    def _():
        acc = jnp.dot(adj_ref[...], zf_ref[...].astype(bf16),
                      preferred_element_type=f32)
        acc = jnp.dot(acc, wh2_ref[...], preferred_element_type=f32)
        t3_ref[pl.ds(i * BM, BM), :] = (
            h1r_ref[...] + jnp.maximum(acc, 0.0)).astype(bf16)

    @pl.when(p == 1)
    def _():
        acc = jnp.dot(adj_ref[...], t3_ref[...],
                      preferred_element_type=f32)
        o_ref[...] = jnp.dot(acc, wout_ref[...], preferred_element_type=f32)


def _head(adj_b, z_f, h1r, wh2, wout):
    return pl.pallas_call(
        _head_body,
        grid=(2, N // BM),
        in_specs=[pl.BlockSpec((BM, N), lambda p, i: (i, 0)),
                  pl.BlockSpec((N, 16), lambda p, i: (0, 0)),
                  pl.BlockSpec((BM, 32), lambda p, i: (i, 0)),
                  pl.BlockSpec((16, 32), lambda p, i: (0, 0)),
                  pl.BlockSpec((32, 16), lambda p, i: (0, 0))],
        out_specs=pl.BlockSpec((BM, 16), lambda p, i: (i, 0)),
        out_shape=jax.ShapeDtypeStruct((N, 16), f32),
        scratch_shapes=[pltpu.VMEM((N, 32), bf16)],
    )(adj_b, z_f, h1r, wh2, wout)


def _dec_body(z_blk, z_all, xw0, w1a, w1b, w2a, w2b,
              zf_ref, rec_ref,
              rs_ref, m_ref, md_ref, vd_ref, u_ref, zfs_ref):
    p = pl.program_id(0)
    i = pl.program_id(1)
    row = pl.ds(i * BM, BM)

    @pl.when(p == 0)
    def _():
        zi = z_blk[...]
        sg = _sig_t(zi, z_all[...])
        rs_ref[row, :] = jnp.sum(sg, axis=1, keepdims=True)
        m_ref[row, :] = jnp.concatenate(
            [jnp.dot(zi, w1a[...], preferred_element_type=f32),
             xw0[..., :32],
             jnp.dot(zi, w1b[...], preferred_element_type=f32),
             xw0[..., 32:]], axis=1)

    @pl.when(p == 1)
    def _():
        @pl.when(i == 0)
        def _():
            md_ref[...] = (m_ref[...] * jax.lax.rsqrt(rs_ref[...])).astype(bf16)

        sg = _sig_t(z_blk[...], z_all[...])
        acc = jnp.dot(sg.astype(bf16), md_ref[...], preferred_element_type=f32)
        di = jax.lax.rsqrt(rs_ref[row, :])
        sc = acc * di
        ua = jnp.maximum(sc[:, 0:32], 0.0) + jnp.maximum(sc[:, 32:64], 0.0)
        ub = jnp.maximum(sc[:, 64:96], 0.0) + jnp.maximum(sc[:, 96:128], 0.0)
        v = jnp.concatenate(
            [jnp.dot(ua, w2a[...], preferred_element_type=f32),
             jnp.dot(ub, w2b[...], preferred_element_type=f32)], axis=1)
        vd_ref[row, :] = (v * di).astype(bf16)

    @pl.when(p == 2)
    def _():
        sg = _sig_t(z_blk[...], z_all[...])
        acc = jnp.dot(sg.astype(bf16), vd_ref[...], preferred_element_type=f32)
        w = acc * jax.lax.rsqrt(rs_ref[row, :])
        upd = (1.0 - _AR) * jnp.concatenate([z_blk[...], z_blk[...]], axis=1) \
            + _AR * w
        u_ref[row, :] = upd[:, :16]
        zfs_ref[row, :] = upd[:, 16:]
        zf_ref[...] = upd[:, 16:]

    @pl.when(p == 3)
    def _():
        zf_ref[...] = zfs_ref[row, :]
        rec_ref[...] = _dot_t(u_ref[row, :], u_ref[...]).reshape(BM * N)


def _decoder(z, xw0, w1a, w1b, w2a, w2b):
    return pl.pallas_call(
        _dec_body,
        grid=(4, N // BM),
        in_specs=[pl.BlockSpec((BM, 16), lambda p, i: (i, 0)),
                  pl.BlockSpec((N, 16), lambda p, i: (0, 0)),
                  pl.BlockSpec((BM, 64), lambda p, i: (i, 0)),
                  pl.BlockSpec((16, 32), lambda p, i: (0, 0)),
                  pl.BlockSpec((16, 32), lambda p, i: (0, 0)),
                  pl.BlockSpec((32, 16), lambda p, i: (0, 0)),
                  pl.BlockSpec((32, 16), lambda p, i: (0, 0))],
        out_specs=[pl.BlockSpec((BM, 16), lambda p, i: (i, 0)),
                   pl.BlockSpec((BM * N,), lambda p, i: (i * (p // 3),))],
        out_shape=[jax.ShapeDtypeStruct((N, 16), f32),
                   jax.ShapeDtypeStruct((N * N,), f32)],
        scratch_shapes=[pltpu.VMEM((N, 1), f32),
                        pltpu.VMEM((N, 128), f32),
                        pltpu.VMEM((N, 128), bf16),
                        pltpu.VMEM((N, 32), bf16),
                        pltpu.VMEM((N, 16), f32),
                        pltpu.VMEM((N, 16), f32)],
    )(z, z, xw0, w1a, w1b, w2a, w2b)


def kernel(features, adj, W_e1, W_mean, W_std, Wl0_a, Wl1_a, Wl2_a,
           Wl0_b, Wl1_b, Wl2_b, W_h1, W_h2, W_out):
    wcat = jnp.concatenate([W_e1, W_h1, Wl0_a, Wl0_b], axis=1)
    p = _proj(features, wcat)

    t1, adj_b = _adj_first(adj, p[:, :64])
    hidden1, h1r = t1[:, :32], t1[:, 32:]
    z = _adj_pass(adj_b, hidden1, post_w=W_mean)

    z_f, reconstructions = _decoder(z, p[:, 64:], Wl1_a, Wl1_b, Wl2_a, Wl2_b)

    outputs = _head(adj_b, z_f, h1r, W_h2, W_out)
    return outputs, reconstructions


# speedup vs baseline: 1.6202x; 1.0423x over previous
import functools

import jax
import jax.numpy as jnp
from jax.experimental import pallas as pl
from jax.experimental.pallas import tpu as pltpu

N = 4096
BM = 256
BA = 512
_AR = 0.5
f32 = jnp.float32
bf16 = jnp.bfloat16


def _dot_t(a, b):
    return jax.lax.dot_general(a, b, (((1,), (1,)), ((), ())),
                               preferred_element_type=f32)


def _sig_t(z_blk, z_all):
    s = jax.lax.dot_general((z_blk * 0.5).astype(bf16), z_all.astype(bf16),
                            (((1,), (1,)), ((), ())),
                            preferred_element_type=f32).astype(bf16)
    half = jnp.asarray(0.5, bf16)
    return half * jnp.tanh(s) + half


def _proj_body(x_ref, w_ref, o_ref):
    o_ref[...] = jnp.dot(x_ref[...], w_ref[...], preferred_element_type=f32)


def _proj(x, w):
    n, k = x.shape
    c = w.shape[1]
    return pl.pallas_call(
        _proj_body,
        grid=(n // BM,),
        in_specs=[pl.BlockSpec((BM, k), lambda i: (i, 0)),
                  pl.BlockSpec((k, c), lambda i: (0, 0))],
        out_specs=pl.BlockSpec((BM, c), lambda i: (i, 0)),
        out_shape=jax.ShapeDtypeStruct((n, c), f32),
    )(x, w)


def _adj_first_body(adj_ref, m_ref, o_ref, ab_ref):
    ab = adj_ref[...].astype(bf16)
    ab_ref[...] = ab
    acc = jnp.dot(ab, m_ref[...].astype(bf16), preferred_element_type=f32)
    o_ref[...] = jnp.maximum(acc, 0.0)


def _adj_first(adj, m):
    n = adj.shape[0]
    c = m.shape[1]
    return pl.pallas_call(
        _adj_first_body,
        grid=(n // BA,),
        in_specs=[pl.BlockSpec((BA, n), lambda i: (i, 0)),
                  pl.BlockSpec((n, c), lambda i: (0, 0))],
        out_specs=[pl.BlockSpec((BA, c), lambda i: (i, 0)),
                   pl.BlockSpec((BA, n), lambda i: (i, 0))],
        out_shape=[jax.ShapeDtypeStruct((n, c), f32),
                   jax.ShapeDtypeStruct((n, n), bf16)],
    )(adj, m)


def _adj_body(do_relu, has_w, *refs):
    adj_ref, m_ref = refs[0], refs[1]
    idx = 2
    w_ref = None
    if has_w:
        w_ref = refs[idx]
        idx += 1
    o_ref = refs[idx]
    acc = jnp.dot(adj_ref[...], m_ref[...].astype(bf16),
                  preferred_element_type=f32)
    if has_w:
        acc = jnp.dot(acc, w_ref[...], preferred_element_type=f32)
    if do_relu:
        acc = jnp.maximum(acc, 0.0)
    o_ref[...] = acc


def _adj_pass(adj_b, m, post_w=None, relu=False):
    n = adj_b.shape[0]
    c = m.shape[1]
    cout = post_w.shape[1] if post_w is not None else c
    ins = [adj_b, m]
    specs = [pl.BlockSpec((BA, n), lambda i: (i, 0)),
             pl.BlockSpec((n, c), lambda i: (0, 0))]
    if post_w is not None:
        ins.append(post_w)
        specs.append(pl.BlockSpec(post_w.shape, lambda i: (0, 0)))
    return pl.pallas_call(
        functools.partial(_adj_body, relu, post_w is not None),
        grid=(n // BA,),
        in_specs=specs,
        out_specs=pl.BlockSpec((BA, cout), lambda i: (i, 0)),
        out_shape=jax.ShapeDtypeStruct((n, cout), f32),
    )(*ins)


def _head_body(adj_ref, zf_ref, h1r_ref, wh2_ref, wout_ref, o_ref, t3_ref):
    p = pl.program_id(0)
    i = pl.program_id(1)

    @pl.when(p == 0)
    def _():
        acc = jnp.dot(adj_ref[...], zf_ref[...].astype(bf16),
                      preferred_element_type=f32)
        acc = jnp.dot(acc, wh2_ref[...], preferred_element_type=f32)
        t3_ref[pl.ds(i * BA, BA), :] = (
            h1r_ref[...] + jnp.maximum(acc, 0.0)).astype(bf16)

    @pl.when(p == 1)
    def _():
        acc = jnp.dot(adj_ref[...], t3_ref[...],
                      preferred_element_type=f32)
        o_ref[...] = jnp.dot(acc, wout_ref[...], preferred_element_type=f32)


def _head(adj_b, z_f, h1r, wh2, wout):
    return pl.pallas_call(
        _head_body,
        grid=(2, N // BA),
        in_specs=[pl.BlockSpec((BA, N), lambda p, i: (i, 0)),
                  pl.BlockSpec((N, 16), lambda p, i: (0, 0)),
                  pl.BlockSpec((BA, 32), lambda p, i: (i, 0)),
                  pl.BlockSpec((16, 32), lambda p, i: (0, 0)),
                  pl.BlockSpec((32, 16), lambda p, i: (0, 0))],
        out_specs=pl.BlockSpec((BA, 16), lambda p, i: (i, 0)),
        out_shape=jax.ShapeDtypeStruct((N, 16), f32),
        scratch_shapes=[pltpu.VMEM((N, 32), bf16)],
    )(adj_b, z_f, h1r, wh2, wout)


def _dec_body(z_blk, z_all, xw0, w1a, w1b, w2a, w2b, ones_ref,
              zf_ref, rec_ref,
              rs_ref, m_ref, md_ref, vd_ref, u_ref, zfs_ref):
    p = pl.program_id(0)
    i = pl.program_id(1)
    row = pl.ds(i * BM, BM)

    @pl.when(p == 0)
    def _():
        zi = z_blk[...]
        sg = _sig_t(zi, z_all[...])
        rs_ref[row, :] = jnp.dot(sg, ones_ref[...],
                                 preferred_element_type=f32)[:, :1]
        m_ref[row, :] = jnp.concatenate(
            [jnp.dot(zi, w1a[...], preferred_element_type=f32),
             xw0[..., :32],
             jnp.dot(zi, w1b[...], preferred_element_type=f32),
             xw0[..., 32:]], axis=1)

    @pl.when(p == 1)
    def _():
        @pl.when(i == 0)
        def _():
            md_ref[...] = (m_ref[...] * jax.lax.rsqrt(rs_ref[...])).astype(bf16)

        sg = _sig_t(z_blk[...], z_all[...])
        acc = jnp.dot(sg, md_ref[...], preferred_element_type=f32)
        di = jax.lax.rsqrt(rs_ref[row, :])
        sc = acc * di
        ua = jnp.maximum(sc[:, 0:32], 0.0) + jnp.maximum(sc[:, 32:64], 0.0)
        ub = jnp.maximum(sc[:, 64:96], 0.0) + jnp.maximum(sc[:, 96:128], 0.0)
        v = jnp.concatenate(
            [jnp.dot(ua, w2a[...], preferred_element_type=f32),
             jnp.dot(ub, w2b[...], preferred_element_type=f32)], axis=1)
        vd_ref[row, :] = (v * di).astype(bf16)

    @pl.when(p == 2)
    def _():
        sg = _sig_t(z_blk[...], z_all[...])
        acc = jnp.dot(sg, vd_ref[...], preferred_element_type=f32)
        w = acc * jax.lax.rsqrt(rs_ref[row, :])
        upd = (1.0 - _AR) * jnp.concatenate([z_blk[...], z_blk[...]], axis=1) \
            + _AR * w
        u_ref[row, :] = upd[:, :16]
        zfs_ref[row, :] = upd[:, 16:]
        zf_ref[...] = upd[:, 16:]

    @pl.when(p == 3)
    def _():
        zf_ref[...] = zfs_ref[row, :]
        rec_ref[...] = _dot_t(u_ref[row, :], u_ref[...]).reshape(BM * N)


def _decoder(z, xw0, w1a, w1b, w2a, w2b):
    return pl.pallas_call(
        _dec_body,
        grid=(4, N // BM),
        in_specs=[pl.BlockSpec((BM, 16), lambda p, i: (i, 0)),
                  pl.BlockSpec((N, 16), lambda p, i: (0, 0)),
                  pl.BlockSpec((BM, 64), lambda p, i: (i, 0)),
                  pl.BlockSpec((16, 32), lambda p, i: (0, 0)),
                  pl.BlockSpec((16, 32), lambda p, i: (0, 0)),
                  pl.BlockSpec((32, 16), lambda p, i: (0, 0)),
                  pl.BlockSpec((32, 16), lambda p, i: (0, 0)),
                  pl.BlockSpec((N, 8), lambda p, i: (0, 0))],
        out_specs=[pl.BlockSpec((BM, 16), lambda p, i: (i, 0)),
                   pl.BlockSpec((BM * N,), lambda p, i: (i * (p // 3),))],
        out_shape=[jax.ShapeDtypeStruct((N, 16), f32),
                   jax.ShapeDtypeStruct((N * N,), f32)],
        scratch_shapes=[pltpu.VMEM((N, 1), f32),
                        pltpu.VMEM((N, 128), f32),
                        pltpu.VMEM((N, 128), bf16),
                        pltpu.VMEM((N, 32), bf16),
                        pltpu.VMEM((N, 16), f32),
                        pltpu.VMEM((N, 16), f32)],
    )(z, z, xw0, w1a, w1b, w2a, w2b,
      jnp.ones((N, 8), bf16))


def kernel(features, adj, W_e1, W_mean, W_std, Wl0_a, Wl1_a, Wl2_a,
           Wl0_b, Wl1_b, Wl2_b, W_h1, W_h2, W_out):
    wcat = jnp.concatenate([W_e1, W_h1, Wl0_a, Wl0_b], axis=1)
    p = _proj(features, wcat)

    t1, adj_b = _adj_first(adj, p[:, :64])
    hidden1, h1r = t1[:, :32], t1[:, 32:]
    z = _adj_pass(adj_b, hidden1, post_w=W_mean)

    z_f, reconstructions = _decoder(z, p[:, 64:], Wl1_a, Wl1_b, Wl2_a, Wl2_b)

    outputs = _head(adj_b, z_f, h1r, W_h2, W_out)
    return outputs, reconstructions
